# baseline (speedup 1.0000x reference)
import jax
import jax.numpy as jnp
from jax.experimental import pallas as pl


_BLK_L = 1024


def _add_kernel(x_ref, pos_ref, o_ref):
    o_ref[...] = x_ref[...] + pos_ref[...][None]


def kernel(x, pos_table):
    B, L, D = x.shape
    blk_l = _BLK_L if L % _BLK_L == 0 else L
    grid = (L // blk_l,)
    return pl.pallas_call(
        _add_kernel,
        grid=grid,
        in_specs=[
            pl.BlockSpec((B, blk_l, D), lambda l: (0, l, 0)),
            pl.BlockSpec((blk_l, D), lambda l: (l, 0)),
        ],
        out_specs=pl.BlockSpec((B, blk_l, D), lambda l: (0, l, 0)),
        out_shape=jax.ShapeDtypeStruct((B, L, D), x.dtype),
    )(x, pos_table[:L])
